# baseline (speedup 1.0000x reference)
import functools
import math

import jax
import jax.numpy as jnp
from jax import lax
from jax.experimental import pallas as pl
from jax.experimental.pallas import tpu as pltpu
from jax.experimental.pallas import tpu_sc as plsc

_NB = 32
_MD = 128
_BR = 512
_S = 2048

_WC = _S - 1
_VL = 4608
_NW = 32
_CHUNK = _VL // _NW


def _bucket_kernel(o_ref):
    x = jax.lax.broadcasted_iota(jnp.int32, (1, _VL), 1)
    d = x - _WC
    rb = jnp.where(d > 0, _NB // 2, 0)
    a = jnp.abs(d)
    af = a.astype(jnp.float32)
    rp_if_large = _MD + jnp.log(af / _MD) / math.log(_MD / _NB) * (_NB - _MD)
    rp_if_large = jnp.minimum(rp_if_large, _MD - 1)
    large = rb.astype(jnp.float32) + rp_if_large
    small = (a + rb).astype(jnp.float32)
    out = jnp.where(a < _MD, small, large)
    o_ref[...] = jnp.clip(out, 0, _NB - 1).astype(jnp.int32)


def _bucket_diag():
    return pl.pallas_call(
        _bucket_kernel,
        out_shape=jax.ShapeDtypeStruct((1, _VL), jnp.int32),
    )()


@functools.partial(
    pl.kernel,
    mesh=plsc.VectorSubcoreMesh(core_axis_name="c", subcore_axis_name="s"),
    out_type=jax.ShapeDtypeStruct((_VL,), jnp.float32),
    scratch_types=[
        pltpu.VMEM((_NB,), jnp.float32),
        pltpu.VMEM((_CHUNK,), jnp.int32),
        pltpu.VMEM((_CHUNK,), jnp.float32),
    ],
)
def _sc_lookup(table_hbm, bucket_hbm, out_hbm, tab_v, idx_v, val_v):
    wid = lax.axis_index("s") * 2 + lax.axis_index("c")
    base = wid * _CHUNK
    pltpu.sync_copy(table_hbm, tab_v)
    pltpu.sync_copy(bucket_hbm.at[pl.ds(base, _CHUNK)], idx_v)
    tab_lo = tab_v[pl.ds(0, 16)]
    tab_hi = tab_v[pl.ds(16, 16)]

    def _lane_gather(t, i):
        return t.at[i].get(mode="promise_in_bounds")

    def body(j, carry):
        idx16 = idx_v[pl.ds(j * 16, 16)]
        lo = _lane_gather(tab_lo, jnp.minimum(idx16, 15))
        hi = _lane_gather(tab_hi, jnp.maximum(idx16 - 16, 0))
        val_v[pl.ds(j * 16, 16)] = jnp.where(idx16 < 16, lo, hi)
        return carry

    lax.fori_loop(0, _CHUNK // 16, body, 0)
    pltpu.sync_copy(val_v, out_hbm.at[pl.ds(base, _CHUNK)])


def _add_bias_kernel(x_ref, vec_ref, o_ref, w_ref, bias_ref):
    r = pl.program_id(0)
    h = pl.program_id(1)

    @pl.when((h == 0) & (r == 0))
    def _():
        v = vec_ref[...]
        w_ref[0:1, :] = v
        for si in range(1, 8):
            w_ref[si:si + 1, :] = jnp.roll(v, si, axis=1)

    @pl.when(h == 0)
    def _():
        for g in range(_BR // 8):
            c = _WC - 8 * g
            rem = c % 128
            ba = (c - rem) - r * _BR
            chunk = w_ref[:, pl.ds(pl.multiple_of(ba, 128), _S + 128)]
            bias_ref[8 * g:8 * g + 8, :] = chunk[:, rem:rem + _S]

    o_ref[...] = x_ref[...] + bias_ref[...]


def _run(x, vec):
    bh, s, _ = x.shape
    grid = (s // _BR, bh)
    return pl.pallas_call(
        _add_bias_kernel,
        grid=grid,
        in_specs=[
            pl.BlockSpec((1, _BR, s), lambda r, hh: (hh, r, 0)),
            pl.BlockSpec((1, _VL), lambda r, hh: (0, 0)),
        ],
        out_specs=pl.BlockSpec((1, _BR, s), lambda r, hh: (hh, r, 0)),
        out_shape=jax.ShapeDtypeStruct((bh, s, s), jnp.float32),
        scratch_shapes=[
            pltpu.VMEM((8, _VL), jnp.float32),
            pltpu.VMEM((_BR, s), jnp.float32),
        ],
        compiler_params=pltpu.CompilerParams(
            dimension_semantics=("parallel", "arbitrary")
        ),
    )(x, vec)


def kernel(attention_scores, bias_table):
    b, h, s, _ = attention_scores.shape
    x = attention_scores.reshape(b * h, s, s)
    bucket = _bucket_diag().reshape(_VL)
    vec = _sc_lookup(bias_table.reshape(_NB), bucket)
    out = _run(x, vec.reshape(1, _VL))
    return out.reshape(b, h, s, s)


# speedup vs baseline: 1.1225x; 1.1225x over previous
import math

import jax
import jax.numpy as jnp
from jax.experimental import pallas as pl
from jax.experimental.pallas import tpu as pltpu

_NB = 32
_MD = 128
_BR = 512
_S = 2048

_WC = _S - 1
_WL = 4352


def _bias_bank():
    si = jax.lax.broadcasted_iota(jnp.int32, (8, _WL), 0)
    x = jax.lax.broadcasted_iota(jnp.int32, (8, _WL), 1)
    d = x - si - _WC
    rb = jnp.where(d > 0, _NB // 2, 0)
    a = jnp.abs(d)
    af = a.astype(jnp.float32)
    rp_if_large = _MD + jnp.log(af / _MD) / math.log(_MD / _NB) * (_NB - _MD)
    rp_if_large = jnp.minimum(rp_if_large, _MD - 1)
    large = rb.astype(jnp.float32) + rp_if_large
    small = (a + rb).astype(jnp.float32)
    out = jnp.where(a < _MD, small, large)
    return jnp.clip(out, 0, _NB - 1).astype(jnp.int32)


def _add_bias_kernel(x_ref, table_ref, o_ref, w_ref, bias_ref):
    r = pl.program_id(0)
    h = pl.program_id(1)

    @pl.when((h == 0) & (r == 0))
    def _():
        bucket = _bias_bank()
        acc = jnp.zeros((8, _WL), jnp.float32)
        for k in range(_NB):
            acc = jnp.where(bucket == k, table_ref[k, 0], acc)
        w_ref[...] = acc

    @pl.when(h == 0)
    def _():
        for g in range(_BR // 8):
            c = _WC - 8 * g
            rem = c % 128
            ba = (c - rem) - r * _BR
            chunk = w_ref[:, pl.ds(pl.multiple_of(ba, 128), _S + 128)]
            bias_ref[8 * g:8 * g + 8, :] = chunk[:, rem:rem + _S]

    o_ref[...] = x_ref[...] + bias_ref[...]


def kernel(attention_scores, bias_table):
    b, h, s, _ = attention_scores.shape
    x = attention_scores.reshape(b * h, s, s)
    grid = (s // _BR, b * h)
    out = pl.pallas_call(
        _add_bias_kernel,
        grid=grid,
        in_specs=[
            pl.BlockSpec((1, _BR, s), lambda r, hh: (hh, r, 0)),
            pl.BlockSpec((_NB, 1), lambda r, hh: (0, 0)),
        ],
        out_specs=pl.BlockSpec((1, _BR, s), lambda r, hh: (hh, r, 0)),
        out_shape=jax.ShapeDtypeStruct((b * h, s, s), jnp.float32),
        scratch_shapes=[
            pltpu.VMEM((8, _WL), jnp.float32),
            pltpu.VMEM((_BR, s), jnp.float32),
        ],
        compiler_params=pltpu.CompilerParams(
            dimension_semantics=("parallel", "arbitrary")
        ),
    )(x, bias_table)
    return out.reshape(b, h, s, s)


# speedup vs baseline: 1.1315x; 1.0080x over previous
import math

import jax
import jax.numpy as jnp
from jax.experimental import pallas as pl
from jax.experimental.pallas import tpu as pltpu

_NB = 32
_MD = 128
_BR = 512
_S = 2048

_WC = _S - 1
_WL = 4352


def _bias_bank():
    si = jax.lax.broadcasted_iota(jnp.int32, (8, _WL), 0)
    x = jax.lax.broadcasted_iota(jnp.int32, (8, _WL), 1)
    d = x - si - _WC
    rb = jnp.where(d > 0, _NB // 2, 0)
    a = jnp.abs(d)
    af = a.astype(jnp.float32)
    rp_if_large = _MD + jnp.log(af / _MD) / math.log(_MD / _NB) * (_NB - _MD)
    rp_if_large = jnp.minimum(rp_if_large, _MD - 1)
    large = rb.astype(jnp.float32) + rp_if_large
    small = (a + rb).astype(jnp.float32)
    out = jnp.where(a < _MD, small, large)
    return jnp.clip(out, 0, _NB - 1).astype(jnp.int32)


def _add_bias_kernel(x_ref, table_ref, o_ref, w_ref, bias_ref):
    r = pl.program_id(0)
    h = pl.program_id(1)

    @pl.when((h == 0) & (r == 0))
    def _():
        bucket = _bias_bank()
        acc = jnp.zeros((8, _WL), jnp.float32)
        for k in range(_NB):
            acc = jnp.where(bucket == k, table_ref[k, 0], acc)
        w_ref[...] = acc

    @pl.when(h == 0)
    def _():
        for g in range(_BR // 8):
            c = _WC - 8 * g
            rem = c % 128
            ba = (c - rem) - r * _BR
            chunk = w_ref[:, pl.ds(pl.multiple_of(ba, 128), _S + 128)]
            sliced = chunk[:, rem:rem + _S]
            bias_ref[8 * g:8 * g + 8, :] = sliced
            o_ref[0, 8 * g:8 * g + 8, :] = x_ref[0, 8 * g:8 * g + 8, :] + sliced

    @pl.when(h != 0)
    def _():
        o_ref[...] = x_ref[...] + bias_ref[...]


def kernel(attention_scores, bias_table):
    b, h, s, _ = attention_scores.shape
    x = attention_scores.reshape(b * h, s, s)
    grid = (s // _BR, b * h)
    out = pl.pallas_call(
        _add_bias_kernel,
        grid=grid,
        in_specs=[
            pl.BlockSpec((1, _BR, s), lambda r, hh: (hh, r, 0)),
            pl.BlockSpec((_NB, 1), lambda r, hh: (0, 0)),
        ],
        out_specs=pl.BlockSpec((1, _BR, s), lambda r, hh: (hh, r, 0)),
        out_shape=jax.ShapeDtypeStruct((b * h, s, s), jnp.float32),
        scratch_shapes=[
            pltpu.VMEM((8, _WL), jnp.float32),
            pltpu.VMEM((_BR, s), jnp.float32),
        ],
        compiler_params=pltpu.CompilerParams(
            dimension_semantics=("parallel", "arbitrary")
        ),
    )(x, bias_table)
    return out.reshape(b, h, s, s)


# speedup vs baseline: 1.1586x; 1.0240x over previous
import math

import jax
import jax.numpy as jnp
from jax.experimental import pallas as pl
from jax.experimental.pallas import tpu as pltpu

_NB = 32
_MD = 128
_BR = 512
_S = 2048

_WC = _S - 1
_WL = 4352


def _bias_bank():
    si = jax.lax.broadcasted_iota(jnp.int32, (8, _WL), 0)
    x = jax.lax.broadcasted_iota(jnp.int32, (8, _WL), 1)
    d = x - si - _WC
    rb = jnp.where(d > 0, _NB // 2, 0)
    a = jnp.abs(d)
    af = a.astype(jnp.float32)
    rp_if_large = _MD + jnp.log(af / _MD) / math.log(_MD / _NB) * (_NB - _MD)
    rp_if_large = jnp.minimum(rp_if_large, _MD - 1)
    large = rb.astype(jnp.float32) + rp_if_large
    small = (a + rb).astype(jnp.float32)
    out = jnp.where(a < _MD, small, large)
    return jnp.clip(out, 0, _NB - 1).astype(jnp.int32)


def _add_bias_kernel(x_ref, table_ref, o_ref, w_ref, bias_ref):
    r = pl.program_id(0)
    h = pl.program_id(1)

    @pl.when((h == 0) & (r == 0))
    def _():
        bucket = _bias_bank()
        acc = jnp.zeros((8, _WL), jnp.float32)
        for k in range(_NB):
            acc = jnp.where(bucket == k, table_ref[k, 0], acc)
        w_ref[...] = acc

    @pl.when(h == 0)
    def _():
        for g in range(_BR // 8):
            c = _WC - 8 * g
            rem = c % 128
            ba = (c - rem) - r * _BR
            chunk = w_ref[:, pl.ds(pl.multiple_of(ba, 128), _S + 128)]
            sliced = chunk[:, rem:rem + _S]
            bias_ref[8 * g:8 * g + 8, :] = sliced
            o_ref[:, 8 * g:8 * g + 8, :] = x_ref[:, 8 * g:8 * g + 8, :] + sliced[None]

    @pl.when(h != 0)
    def _():
        o_ref[...] = x_ref[...] + bias_ref[...]


def kernel(attention_scores, bias_table):
    b, h, s, _ = attention_scores.shape
    x = attention_scores.reshape(b * h, s, s)
    hb = 2
    grid = (s // _BR, (b * h) // hb)
    out = pl.pallas_call(
        _add_bias_kernel,
        grid=grid,
        in_specs=[
            pl.BlockSpec((hb, _BR, s), lambda r, hh: (hh, r, 0)),
            pl.BlockSpec((_NB, 1), lambda r, hh: (0, 0)),
        ],
        out_specs=pl.BlockSpec((hb, _BR, s), lambda r, hh: (hh, r, 0)),
        out_shape=jax.ShapeDtypeStruct((b * h, s, s), jnp.float32),
        scratch_shapes=[
            pltpu.VMEM((8, _WL), jnp.float32),
            pltpu.VMEM((_BR, s), jnp.float32),
        ],
        compiler_params=pltpu.CompilerParams(
            dimension_semantics=("parallel", "arbitrary")
        ),
    )(x, bias_table)
    return out.reshape(b, h, s, s)


# speedup vs baseline: 1.1633x; 1.0040x over previous
import math

import jax
import jax.numpy as jnp
from jax.experimental import pallas as pl
from jax.experimental.pallas import tpu as pltpu

_NB = 32
_MD = 128
_BR = 256
_S = 2048

_WC = _S - 1
_WL = 4352


def _bias_bank():
    si = jax.lax.broadcasted_iota(jnp.int32, (8, _WL), 0)
    x = jax.lax.broadcasted_iota(jnp.int32, (8, _WL), 1)
    d = x - si - _WC
    rb = jnp.where(d > 0, _NB // 2, 0)
    a = jnp.abs(d)
    af = a.astype(jnp.float32)
    rp_if_large = _MD + jnp.log(af / _MD) / math.log(_MD / _NB) * (_NB - _MD)
    rp_if_large = jnp.minimum(rp_if_large, _MD - 1)
    large = rb.astype(jnp.float32) + rp_if_large
    small = (a + rb).astype(jnp.float32)
    out = jnp.where(a < _MD, small, large)
    return jnp.clip(out, 0, _NB - 1).astype(jnp.int32)


def _add_bias_kernel(x_ref, table_ref, o_ref, w_ref, bias_ref):
    r = pl.program_id(0)
    h = pl.program_id(1)

    @pl.when((h == 0) & (r == 0))
    def _():
        bucket = _bias_bank()
        acc = jnp.zeros((8, _WL), jnp.float32)
        for k in range(_NB):
            acc = jnp.where(bucket == k, table_ref[k, 0], acc)
        w_ref[...] = acc

    @pl.when(h == 0)
    def _():
        for g in range(_BR // 8):
            c = _WC - 8 * g
            rem = c % 128
            ba = (c - rem) - r * _BR
            chunk = w_ref[:, pl.ds(pl.multiple_of(ba, 128), _S + 128)]
            sliced = chunk[:, rem:rem + _S]
            bias_ref[8 * g:8 * g + 8, :] = sliced
            o_ref[:, 8 * g:8 * g + 8, :] = x_ref[:, 8 * g:8 * g + 8, :] + sliced[None]

    @pl.when(h != 0)
    def _():
        o_ref[...] = x_ref[...] + bias_ref[...]


def kernel(attention_scores, bias_table):
    b, h, s, _ = attention_scores.shape
    x = attention_scores.reshape(b * h, s, s)
    hb = 4
    grid = (s // _BR, (b * h) // hb)
    out = pl.pallas_call(
        _add_bias_kernel,
        grid=grid,
        in_specs=[
            pl.BlockSpec((hb, _BR, s), lambda r, hh: (hh, r, 0)),
            pl.BlockSpec((_NB, 1), lambda r, hh: (0, 0)),
        ],
        out_specs=pl.BlockSpec((hb, _BR, s), lambda r, hh: (hh, r, 0)),
        out_shape=jax.ShapeDtypeStruct((b * h, s, s), jnp.float32),
        scratch_shapes=[
            pltpu.VMEM((8, _WL), jnp.float32),
            pltpu.VMEM((_BR, s), jnp.float32),
        ],
        compiler_params=pltpu.CompilerParams(
            dimension_semantics=("parallel", "arbitrary")
        ),
    )(x, bias_table)
    return out.reshape(b, h, s, s)
